# Initial kernel scaffold; baseline (speedup 1.0000x reference)
import jax
import jax.numpy as jnp
from jax.experimental import pallas as pl


def kernel(inputs, emb_table, W, b):
    raise NotImplementedError("write your pallas kernel here")


# speedup vs baseline: 4.8328x; 4.8328x over previous
import functools

import jax
import jax.numpy as jnp
from jax import lax
from jax.experimental import pallas as pl
from jax.experimental.pallas import tpu as pltpu
from jax.experimental.pallas import tpu_sc as plsc

NUM_CLASSES = 1000
DIM_EMB = 11
DENSE_OUT = 5

_NC = 2
_NS = 16
_NW = _NC * _NS
_LANES = 16


def _proj_body(emb_ref, w_ref, b_ref, out_ref):
    out_ref[...] = jnp.tanh(
        jnp.dot(emb_ref[...], w_ref[...], preferred_element_type=jnp.float32)
        + b_ref[...]
    )


def _project_table(emb_table, W, b):
    return pl.pallas_call(
        _proj_body,
        out_shape=jax.ShapeDtypeStruct((NUM_CLASSES, DENSE_OUT), jnp.float32),
    )(emb_table, W, b.reshape(1, DENSE_OUT))


@functools.cache
def _make_gather(total: int, blk: int):
    per_w = total // _NW
    nblk = per_w // blk
    grp_per_blk = blk // _LANES

    mesh = plsc.VectorSubcoreMesh(core_axis_name="c", subcore_axis_name="s")

    @functools.partial(
        pl.kernel,
        mesh=mesh,
        compiler_params=pltpu.CompilerParams(needs_layout_passes=False),
        out_type=jax.ShapeDtypeStruct((total * DENSE_OUT,), jnp.float32),
        scratch_types=[
            pltpu.VMEM((NUM_CLASSES * DENSE_OUT,), jnp.float32),
            pltpu.VMEM((blk,), jnp.int32),
            pltpu.VMEM((blk * DENSE_OUT,), jnp.float32),
        ],
    )
    def gather_kernel(table_hbm, idx_hbm, out_hbm, table_v, idx_v, out_v):
        wid = lax.axis_index("s") * _NC + lax.axis_index("c")
        pltpu.sync_copy(table_hbm, table_v)
        iota = lax.iota(jnp.int32, _LANES)

        def blk_body(bi, _):
            base = wid * per_w + bi * blk
            pltpu.sync_copy(idx_hbm.at[pl.ds(base, blk)], idx_v)

            def grp(g, _):
                iv = idx_v[pl.ds(g * _LANES, _LANES)]
                addr = iv * DENSE_OUT
                pos = (g * _LANES + iota) * DENSE_OUT
                for c in range(DENSE_OUT):
                    v = plsc.load_gather(table_v, [addr + c])
                    plsc.store_scatter(out_v, [pos + c], v)
                return 0

            lax.fori_loop(0, grp_per_blk, grp, 0, unroll=4)
            pltpu.sync_copy(
                out_v, out_hbm.at[pl.ds(base * DENSE_OUT, blk * DENSE_OUT)]
            )
            return 0

        lax.fori_loop(0, nblk, blk_body, 0)

    return gather_kernel


def kernel(inputs, emb_table, W, b):
    batch, seqlen = inputs.shape
    total = batch * seqlen
    assert total % (_NW * _LANES) == 0

    blk = 2048
    while total % (_NW * blk) != 0:
        blk //= 2

    proj = _project_table(emb_table, W, b)
    table_flat = proj.reshape(-1)
    idx_flat = inputs.reshape(-1).astype(jnp.int32)
    out_flat = _make_gather(total, blk)(table_flat, idx_flat)
    return out_flat.reshape(batch, seqlen, DENSE_OUT)


# speedup vs baseline: 47.0582x; 9.7373x over previous
import functools

import jax
import jax.numpy as jnp
from jax import lax
from jax.experimental import pallas as pl
from jax.experimental.pallas import tpu as pltpu
from jax.experimental.pallas import tpu_sc as plsc

NUM_CLASSES = 1000
DIM_EMB = 11
DENSE_OUT = 5

_NC = 2
_NS = 16
_NW = _NC * _NS
_LANES = 16


def _proj_body(emb_ref, w_ref, b_ref, out_ref):
    out_ref[...] = jnp.tanh(
        jnp.dot(emb_ref[...], w_ref[...], preferred_element_type=jnp.float32)
        + b_ref[...]
    )


def _project_table(emb_table, W, b):
    return pl.pallas_call(
        _proj_body,
        out_shape=jax.ShapeDtypeStruct((NUM_CLASSES, DENSE_OUT), jnp.float32),
    )(emb_table, W, b.reshape(1, DENSE_OUT))


@functools.cache
def _make_gather(total: int, blk: int):
    per_w = total // _NW
    nblk = per_w // blk
    grp_per_blk = blk // _LANES

    mesh = plsc.VectorSubcoreMesh(core_axis_name="c", subcore_axis_name="s")

    @functools.partial(
        pl.kernel,
        mesh=mesh,
        compiler_params=pltpu.CompilerParams(needs_layout_passes=False),
        out_type=jax.ShapeDtypeStruct((DENSE_OUT * total,), jnp.float32),
        scratch_types=[
            pltpu.VMEM((NUM_CLASSES * DENSE_OUT,), jnp.float32),
            pltpu.VMEM((blk,), jnp.int32),
            pltpu.VMEM((DENSE_OUT * blk,), jnp.float32),
        ],
    )
    def gather_kernel(table_hbm, idx_hbm, out_hbm, table_v, idx_v, out_v):
        wid = lax.axis_index("s") * _NC + lax.axis_index("c")
        pltpu.sync_copy(table_hbm, table_v)

        def blk_body(bi, _):
            base = wid * per_w + bi * blk
            pltpu.sync_copy(idx_hbm.at[pl.ds(base, blk)], idx_v)

            def grp(g, _):
                iv = idx_v[pl.ds(g * _LANES, _LANES)]
                addr = iv * DENSE_OUT
                for c in range(DENSE_OUT):
                    v = plsc.load_gather(table_v, [addr + c])
                    out_v[pl.ds(c * blk + g * _LANES, _LANES)] = v
                return 0

            lax.fori_loop(0, grp_per_blk, grp, 0, unroll=4)
            for c in range(DENSE_OUT):
                pltpu.sync_copy(
                    out_v.at[pl.ds(c * blk, blk)],
                    out_hbm.at[pl.ds(c * total + base, blk)],
                )
            return 0

        lax.fori_loop(0, nblk, blk_body, 0)

    return gather_kernel


def kernel(inputs, emb_table, W, b):
    batch, seqlen = inputs.shape
    total = batch * seqlen
    assert batch % 128 == 0 and seqlen % 8 == 0

    blk = 2048
    while total % (_NW * blk) != 0:
        blk //= 2

    proj = _project_table(emb_table, W, b)
    table_flat = proj.reshape(-1)

    tb, tl = batch // 128, seqlen // 8
    idx_phys = (
        inputs.astype(jnp.int32)
        .reshape(tb, 128, tl, 8)
        .transpose(2, 0, 3, 1)
        .reshape(-1)
    )

    out_flat = _make_gather(total, blk)(table_flat, idx_phys)

    out = (
        out_flat.reshape(DENSE_OUT, tl, tb, 8, 128)
        .transpose(2, 4, 1, 3, 0)
        .reshape(batch, seqlen, DENSE_OUT)
    )
    return out


# speedup vs baseline: 67.7861x; 1.4405x over previous
import functools

import jax
import jax.numpy as jnp
from jax import lax
from jax.experimental import pallas as pl
from jax.experimental.pallas import tpu as pltpu
from jax.experimental.pallas import tpu_sc as plsc

NUM_CLASSES = 1000
DIM_EMB = 11
DENSE_OUT = 5
_TPAD = 1024

_NC = 2
_NS = 16
_NW = _NC * _NS
_LANES = 16


def _proj_body(embt_ref, wt_ref, b_ref, out_ref):
    out_ref[...] = jnp.tanh(
        jnp.dot(wt_ref[...], embt_ref[...], preferred_element_type=jnp.float32)
        + b_ref[...]
    )


def _project_table(emb_table, W, b):
    embt = jnp.pad(emb_table.T, ((0, 0), (0, _TPAD - NUM_CLASSES)))
    table = pl.pallas_call(
        _proj_body,
        out_shape=jax.ShapeDtypeStruct((DENSE_OUT, _TPAD), jnp.float32),
    )(embt, W.T, b.reshape(DENSE_OUT, 1))
    return table.reshape(-1)


@functools.cache
def _make_gather(total: int, blk: int):
    per_w = total // _NW
    nblk = per_w // blk
    assert nblk % 2 == 0
    grp_per_blk = blk // _LANES

    mesh = plsc.VectorSubcoreMesh(core_axis_name="c", subcore_axis_name="s")

    @functools.partial(
        pl.kernel,
        mesh=mesh,
        compiler_params=pltpu.CompilerParams(needs_layout_passes=False),
        out_type=jax.ShapeDtypeStruct((DENSE_OUT * total,), jnp.float32),
        scratch_types=[
            pltpu.VMEM((DENSE_OUT * _TPAD,), jnp.float32),
            pltpu.VMEM((blk,), jnp.int32),
            pltpu.VMEM((blk,), jnp.int32),
            pltpu.VMEM((DENSE_OUT * blk,), jnp.float32),
            pltpu.VMEM((DENSE_OUT * blk,), jnp.float32),
            pltpu.SemaphoreType.DMA,
            pltpu.SemaphoreType.DMA,
            pltpu.SemaphoreType.DMA,
            pltpu.SemaphoreType.DMA,
        ],
    )
    def gather_kernel(
        table_hbm, idx_hbm, out_hbm,
        table_v, i0, i1, o0, o1, si0, si1, so0, so1,
    ):
        wid = lax.axis_index("s") * _NC + lax.axis_index("c")
        wbase = wid * per_w
        ibufs, obufs = (i0, i1), (o0, o1)
        isems, osems = (si0, si1), (so0, so1)
        pltpu.sync_copy(table_hbm, table_v)

        def idx_src(b):
            return idx_hbm.at[pl.ds(wbase + b * blk, blk)]

        def out_dst(b, c):
            return out_hbm.at[pl.ds(c * total + wbase + b * blk, blk)]

        pltpu.async_copy(idx_src(0), ibufs[0], isems[0])

        def pair_body(pi, _):
            for p in range(2):
                b = pi * 2 + p
                ib, ob = ibufs[p], obufs[p]

                @pl.when(b + 1 < nblk)
                def _():
                    pltpu.async_copy(idx_src(b + 1), ibufs[1 - p], isems[1 - p])

                pltpu.make_async_copy(idx_src(b), ib, isems[p]).wait()

                @pl.when(b >= 2)
                def _():
                    for c in range(DENSE_OUT):
                        pltpu.make_async_copy(
                            ob.at[pl.ds(c * blk, blk)], out_dst(b, c), osems[p]
                        ).wait()

                def grp(g, _):
                    iv = ib[pl.ds(g * _LANES, _LANES)]
                    for c in range(DENSE_OUT):
                        v = plsc.load_gather(table_v, [iv + c * _TPAD])
                        ob[pl.ds(c * blk + g * _LANES, _LANES)] = v
                    return 0

                lax.fori_loop(0, grp_per_blk, grp, 0, unroll=4)
                for c in range(DENSE_OUT):
                    pltpu.async_copy(
                        ob.at[pl.ds(c * blk, blk)], out_dst(b, c), osems[p]
                    )
            return 0

        lax.fori_loop(0, nblk // 2, pair_body, 0)
        for p in range(2):
            for c in range(DENSE_OUT):
                pltpu.make_async_copy(
                    obufs[p].at[pl.ds(c * blk, blk)],
                    out_dst(nblk - 2 + p, c),
                    osems[p],
                ).wait()

    return gather_kernel


def kernel(inputs, emb_table, W, b):
    batch, seqlen = inputs.shape
    total = batch * seqlen
    assert batch % 128 == 0 and seqlen % 8 == 0

    blk = 2048
    while total % (_NW * blk) != 0 or (total // (_NW * blk)) % 2 != 0:
        blk //= 2

    table = _project_table(emb_table, W, b)

    tb, tl = batch // 128, seqlen // 8
    idx_phys = (
        inputs.astype(jnp.int32)
        .reshape(tb, 128, tl, 8)
        .transpose(2, 0, 3, 1)
        .reshape(-1)
    )

    out_flat = _make_gather(total, blk)(table, idx_phys)

    out = (
        out_flat.reshape(DENSE_OUT, tl, tb, 8, 128)
        .transpose(2, 4, 1, 3, 0)
        .reshape(batch, seqlen, DENSE_OUT)
    )
    return out
